# Initial kernel scaffold; baseline (speedup 1.0000x reference)
import jax
import jax.numpy as jnp
from jax.experimental import pallas as pl


def kernel(x, adj, W1, b1, W2, b2, W3, b3, W4, b4):
    raise NotImplementedError("write your pallas kernel here")


# speedup vs baseline: 22.8414x; 22.8414x over previous
import functools

import jax
import jax.numpy as jnp
from jax import lax
from jax.experimental import pallas as pl
from jax.experimental.pallas import tpu as pltpu
from jax.experimental.pallas import tpu_sc as plsc

N = 10000
E = 320000
NC, NS = 2, 16
NW = NC * NS
EPW = E // NW
CH = 80
NCHUNK = EPW // CH
RPT = N // NS

_MESH = plsc.VectorSubcoreMesh(core_axis_name="c", subcore_axis_name="s")


@functools.partial(
    pl.kernel,
    out_type=jax.ShapeDtypeStruct((NW, N), jnp.float32),
    mesh=_MESH,
    scratch_types=[
        pltpu.VMEM((N,), jnp.float32),
        pltpu.VMEM((EPW,), jnp.int32),
    ],
    compiler_params=pltpu.CompilerParams(needs_layout_passes=False),
)
def _deg(dst_hbm, degp_hbm, deg_v, didx_v):
    cid = lax.axis_index("c")
    sid = lax.axis_index("s")
    wid = sid * NC + cid

    zeros = jnp.zeros((16,), jnp.float32)

    def zbody(i, c):
        deg_v[pl.ds(i * 16, 16)] = zeros
        return c

    lax.fori_loop(0, N // 16, zbody, 0)

    pltpu.sync_copy(dst_hbm.at[pl.ds(wid * EPW, EPW)], didx_v)

    ones = jnp.full((16,), 1.0, jnp.float32)

    def ebody(i, c):
        idx = didx_v[pl.ds(i * 16, 16)]
        plsc.addupdate_scatter(deg_v, [idx], ones)
        return c

    lax.fori_loop(0, EPW // 16, ebody, 0)

    pltpu.sync_copy(deg_v, degp_hbm.at[wid])


def _make_agg(F):
    @functools.partial(
        pl.kernel,
        out_type=jax.ShapeDtypeStruct((NC, N, F), jnp.float32),
        mesh=_MESH,
        scratch_types=[
            pltpu.VMEM_SHARED((N, F), jnp.float32),
            pltpu.VMEM((NCHUNK, CH), jnp.int32),
            pltpu.VMEM((NCHUNK, CH), jnp.int32),
            pltpu.VMEM((CH, F), jnp.float32),
            pltpu.VMEM((RPT, F), jnp.float32),
            pltpu.SemaphoreType.DMA,
            pltpu.SemaphoreType.DMA,
        ],
        compiler_params=pltpu.CompilerParams(use_tc_tiling_on_sc=False),
    )
    def agg(u_hbm, src_hbm, dst_hbm, p_hbm, acc, sidx, didx, rows, stage,
            gsem, ssem):
        cid = lax.axis_index("c")
        sid = lax.axis_index("s")
        wid = sid * NC + cid
        rbase = sid * RPT

        pltpu.sync_copy(u_hbm.at[pl.ds(rbase, RPT)], stage)
        pltpu.sync_copy(stage, acc.at[pl.ds(rbase, RPT)])

        pltpu.sync_copy(src_hbm.at[wid], sidx)
        pltpu.sync_copy(dst_hbm.at[wid], didx)

        plsc.subcore_barrier()

        def body(j, c):
            pltpu.async_copy(u_hbm.at[sidx.at[j]], rows, gsem).wait()
            pltpu.async_copy(rows, acc.at[didx.at[j]], ssem, add=True).wait()
            return c

        lax.fori_loop(0, NCHUNK, body, 0)

        plsc.subcore_barrier()

        pltpu.sync_copy(acc.at[pl.ds(rbase, RPT)], stage)
        pltpu.sync_copy(stage, p_hbm.at[cid].at[pl.ds(rbase, RPT)])

    return agg


_agg8 = _make_agg(8)
_agg16 = _make_agg(16)
_agg40 = _make_agg(40)


def _tc_first_body(degpT_ref, x_ref, w_ref, dinv_ref, u_ref):
    deg = jnp.sum(degpT_ref[...], axis=1, keepdims=True) + 1.0
    dinv = lax.rsqrt(deg)
    dinv_ref[...] = dinv
    u_ref[...] = dinv * jnp.dot(x_ref[...], w_ref[...],
                                preferred_element_type=jnp.float32)


def _tc_mid_body(p_ref, u_ref, dinv_ref, b_ref, w_ref, un_ref):
    s = p_ref[0] + p_ref[1] - u_ref[...]
    pre = dinv_ref[...] * s + b_ref[...]
    h = jnp.maximum(pre, 0.0)
    un_ref[...] = dinv_ref[...] * jnp.dot(h, w_ref[...],
                                          preferred_element_type=jnp.float32)


def _tc_last_body(p_ref, u_ref, dinv_ref, b_ref, o_ref):
    s = p_ref[0] + p_ref[1] - u_ref[...]
    pre = dinv_ref[...] * s + b_ref[...]
    m = jnp.max(pre, axis=1, keepdims=True)
    lse = jnp.log(jnp.sum(jnp.exp(pre - m), axis=1, keepdims=True)) + m
    o_ref[...] = pre - lse


def _tc_first(degpT, x, w):
    return pl.pallas_call(
        _tc_first_body,
        out_shape=(
            jax.ShapeDtypeStruct((N, 1), jnp.float32),
            jax.ShapeDtypeStruct((N, w.shape[1]), jnp.float32),
        ),
    )(degpT, x, w)


def _tc_mid(p, u, dinv, b, w):
    return pl.pallas_call(
        _tc_mid_body,
        out_shape=jax.ShapeDtypeStruct((N, w.shape[1]), jnp.float32),
    )(p, u, dinv, b, w)


def _tc_last(p, u, dinv, b):
    return pl.pallas_call(
        _tc_last_body,
        out_shape=jax.ShapeDtypeStruct((N, u.shape[1]), jnp.float32),
    )(p, u, dinv, b)


def kernel(x, adj, W1, b1, W2, b2, W3, b3, W4, b4):
    src = adj[0].astype(jnp.int32)
    dst = adj[1].astype(jnp.int32)
    src3 = src.reshape(NW, NCHUNK, CH)
    dst3 = dst.reshape(NW, NCHUNK, CH)

    degp = _deg(dst)
    degpT = degp.T

    dinv, u1 = _tc_first(degpT, x, W1)

    p1 = _agg8(u1, src3, dst3)
    u2 = _tc_mid(p1, u1, dinv, b1.reshape(1, -1), W2)

    p2 = _agg16(u2, src3, dst3)
    u3 = _tc_mid(p2, u2, dinv, b2.reshape(1, -1), W3)

    p3 = _agg8(u3, src3, dst3)
    u4 = _tc_mid(p3, u3, dinv, b3.reshape(1, -1), W4)

    p4 = _agg40(u4, src3, dst3)
    return _tc_last(p4, u4, dinv, b4.reshape(1, -1))


# speedup vs baseline: 45.8710x; 2.0082x over previous
import functools

import jax
import jax.numpy as jnp
from jax import lax
from jax.experimental import pallas as pl
from jax.experimental.pallas import tpu as pltpu
from jax.experimental.pallas import tpu_sc as plsc

N = 10000
E = 320000
NC, NS = 2, 16
NW = NC * NS
EPW = E // NW
CH = 80
NCHUNK = EPW // CH
RPT = N // NS

_MESH = plsc.VectorSubcoreMesh(core_axis_name="c", subcore_axis_name="s")


@functools.partial(
    pl.kernel,
    out_type=jax.ShapeDtypeStruct((NW, N), jnp.float32),
    mesh=_MESH,
    scratch_types=[
        pltpu.VMEM((N,), jnp.float32),
        pltpu.VMEM((EPW,), jnp.int32),
    ],
    compiler_params=pltpu.CompilerParams(needs_layout_passes=False),
)
def _deg(dst_hbm, degp_hbm, deg_v, didx_v):
    cid = lax.axis_index("c")
    sid = lax.axis_index("s")
    wid = sid * NC + cid

    zeros = jnp.zeros((16,), jnp.float32)

    def zbody(i, c):
        deg_v[pl.ds(i * 16, 16)] = zeros
        return c

    lax.fori_loop(0, N // 16, zbody, 0)

    pltpu.sync_copy(dst_hbm.at[pl.ds(wid * EPW, EPW)], didx_v)

    ones = jnp.full((16,), 1.0, jnp.float32)

    def ebody(i, c):
        idx = didx_v[pl.ds(i * 16, 16)]
        plsc.addupdate_scatter(deg_v, [idx], ones)
        return c

    lax.fori_loop(0, EPW // 16, ebody, 0)

    pltpu.sync_copy(deg_v, degp_hbm.at[wid])


NBUF = 5
GRP = NCHUNK // NBUF


def _make_agg(F):
    @functools.partial(
        pl.kernel,
        out_type=jax.ShapeDtypeStruct((NC, N, F), jnp.float32),
        mesh=_MESH,
        scratch_types=(
            [
                pltpu.VMEM_SHARED((N, F), jnp.float32),
                pltpu.VMEM((NCHUNK, CH), jnp.int32),
                pltpu.VMEM((NCHUNK, CH), jnp.int32),
                pltpu.VMEM((RPT, F), jnp.float32),
            ]
            + [pltpu.VMEM((CH, F), jnp.float32) for _ in range(NBUF)]
            + [pltpu.SemaphoreType.DMA for _ in range(2 * NBUF)]
        ),
        compiler_params=pltpu.CompilerParams(use_tc_tiling_on_sc=False),
    )
    def agg(u_hbm, src_hbm, dst_hbm, p_hbm, acc, sidx, didx, stage, *bufs_sems):
        rows = bufs_sems[:NBUF]
        gs = bufs_sems[NBUF:2 * NBUF]
        ss = bufs_sems[2 * NBUF:]
        cid = lax.axis_index("c")
        sid = lax.axis_index("s")
        wid = sid * NC + cid
        rbase = sid * RPT

        pltpu.sync_copy(u_hbm.at[pl.ds(rbase, RPT)], stage)
        pltpu.sync_copy(stage, acc.at[pl.ds(rbase, RPT)])

        pltpu.sync_copy(src_hbm.at[wid], sidx)
        pltpu.sync_copy(dst_hbm.at[wid], didx)

        plsc.subcore_barrier()

        for b in range(NBUF):
            pltpu.async_copy(u_hbm.at[sidx.at[b]], rows[b], gs[b])

        def round_(g, c):
            for b in range(NBUF):
                j = g * NBUF + b
                pltpu.make_async_copy(u_hbm.at[sidx.at[j]], rows[b], gs[b]).wait()
                pltpu.async_copy(rows[b], acc.at[didx.at[j]], ss[b], add=True)
            for b in range(NBUF):
                pltpu.make_async_copy(rows[b], acc.at[didx.at[b]], ss[b]).wait()

                @pl.when(g + 1 < GRP)
                def _():
                    jn = (g + 1) * NBUF + b
                    pltpu.async_copy(u_hbm.at[sidx.at[jn]], rows[b], gs[b])

            return c

        lax.fori_loop(0, GRP, round_, 0)

        plsc.subcore_barrier()

        pltpu.sync_copy(acc.at[pl.ds(rbase, RPT)], stage)
        pltpu.sync_copy(stage, p_hbm.at[cid].at[pl.ds(rbase, RPT)])

    return agg


_agg8 = _make_agg(8)
_agg16 = _make_agg(16)
_agg40 = _make_agg(40)


def _tc_first_body(degpT_ref, x_ref, w_ref, dinv_ref, u_ref):
    deg = jnp.sum(degpT_ref[...], axis=1, keepdims=True) + 1.0
    dinv = lax.rsqrt(deg)
    dinv_ref[...] = dinv
    u_ref[...] = dinv * jnp.dot(x_ref[...], w_ref[...],
                                preferred_element_type=jnp.float32)


def _tc_mid_body(p_ref, u_ref, dinv_ref, b_ref, w_ref, un_ref):
    s = p_ref[0] + p_ref[1] - u_ref[...]
    pre = dinv_ref[...] * s + b_ref[...]
    h = jnp.maximum(pre, 0.0)
    un_ref[...] = dinv_ref[...] * jnp.dot(h, w_ref[...],
                                          preferred_element_type=jnp.float32)


def _tc_last_body(p_ref, u_ref, dinv_ref, b_ref, o_ref):
    s = p_ref[0] + p_ref[1] - u_ref[...]
    pre = dinv_ref[...] * s + b_ref[...]
    m = jnp.max(pre, axis=1, keepdims=True)
    lse = jnp.log(jnp.sum(jnp.exp(pre - m), axis=1, keepdims=True)) + m
    o_ref[...] = pre - lse


def _tc_first(degpT, x, w):
    return pl.pallas_call(
        _tc_first_body,
        out_shape=(
            jax.ShapeDtypeStruct((N, 1), jnp.float32),
            jax.ShapeDtypeStruct((N, w.shape[1]), jnp.float32),
        ),
    )(degpT, x, w)


def _tc_mid(p, u, dinv, b, w):
    return pl.pallas_call(
        _tc_mid_body,
        out_shape=jax.ShapeDtypeStruct((N, w.shape[1]), jnp.float32),
    )(p, u, dinv, b, w)


def _tc_last(p, u, dinv, b):
    return pl.pallas_call(
        _tc_last_body,
        out_shape=jax.ShapeDtypeStruct((N, u.shape[1]), jnp.float32),
    )(p, u, dinv, b)


def kernel(x, adj, W1, b1, W2, b2, W3, b3, W4, b4):
    src = adj[0].astype(jnp.int32)
    dst = adj[1].astype(jnp.int32)
    src3 = src.reshape(NW, NCHUNK, CH)
    dst3 = dst.reshape(NW, NCHUNK, CH)

    degp = _deg(dst)
    degpT = degp.T

    dinv, u1 = _tc_first(degpT, x, W1)

    p1 = _agg8(u1, src3, dst3)
    u2 = _tc_mid(p1, u1, dinv, b1.reshape(1, -1), W2)

    p2 = _agg16(u2, src3, dst3)
    u3 = _tc_mid(p2, u2, dinv, b2.reshape(1, -1), W3)

    p3 = _agg8(u3, src3, dst3)
    u4 = _tc_mid(p3, u3, dinv, b3.reshape(1, -1), W4)

    p4 = _agg40(u4, src3, dst3)
    return _tc_last(p4, u4, dinv, b4.reshape(1, -1))


# speedup vs baseline: 49.0928x; 1.0702x over previous
import functools

import jax
import jax.numpy as jnp
from jax import lax
from jax.experimental import pallas as pl
from jax.experimental.pallas import tpu as pltpu
from jax.experimental.pallas import tpu_sc as plsc

N = 10000
N_EXT = 10112
E = 320000
NC, NS = 2, 16
NW = NC * NS
CH = 128
NCHUNK = 80
EPT = NCHUNK * CH
E_PAD = NW * EPT
ROWS2D = E_PAD // CH
RPT = N_EXT // NS
NBUF = 5
GRP = NCHUNK // NBUF

_MESH = plsc.VectorSubcoreMesh(core_axis_name="c", subcore_axis_name="s")


@functools.partial(
    pl.kernel,
    out_type=jax.ShapeDtypeStruct((NW, N_EXT), jnp.float32),
    mesh=_MESH,
    scratch_types=[
        pltpu.VMEM((N_EXT,), jnp.float32),
        pltpu.VMEM((NCHUNK, CH), jnp.int32),
    ],
    compiler_params=pltpu.CompilerParams(
        needs_layout_passes=False, use_tc_tiling_on_sc=False),
)
def _deg(dst_hbm, degp_hbm, deg_v, didx_v):
    cid = lax.axis_index("c")
    sid = lax.axis_index("s")
    wid = sid * NC + cid

    zeros = jnp.zeros((16,), jnp.float32)

    def zbody(i, c):
        deg_v[pl.ds(i * 16, 16)] = zeros
        return c

    lax.fori_loop(0, N_EXT // 16, zbody, 0)

    pltpu.sync_copy(dst_hbm.at[pl.ds(wid * NCHUNK, NCHUNK)], didx_v)

    ones = jnp.full((16,), 1.0, jnp.float32)

    def ebody(i, c):
        idx = didx_v[i // (CH // 16), pl.ds((i % (CH // 16)) * 16, 16)]
        plsc.addupdate_scatter(deg_v, [idx], ones)
        return c

    lax.fori_loop(0, EPT // 16, ebody, 0)

    pltpu.sync_copy(deg_v, degp_hbm.at[wid])


def _make_agg(F):
    @functools.partial(
        pl.kernel,
        out_type=jax.ShapeDtypeStruct((NC, N_EXT, F), jnp.float32),
        mesh=_MESH,
        scratch_types=(
            [
                pltpu.VMEM_SHARED((N_EXT, F), jnp.float32),
                pltpu.VMEM((NCHUNK, CH), jnp.int32),
                pltpu.VMEM((NCHUNK, CH), jnp.int32),
                pltpu.VMEM((RPT, F), jnp.float32),
            ]
            + [pltpu.VMEM((CH, F), jnp.float32) for _ in range(NBUF)]
            + [pltpu.SemaphoreType.DMA for _ in range(2 * NBUF)]
        ),
        compiler_params=pltpu.CompilerParams(use_tc_tiling_on_sc=False),
    )
    def agg(u_hbm, src_hbm, dst_hbm, p_hbm, acc, sidx, didx, stage, *bufs_sems):
        rows = bufs_sems[:NBUF]
        gs = bufs_sems[NBUF:2 * NBUF]
        ss = bufs_sems[2 * NBUF:]
        cid = lax.axis_index("c")
        sid = lax.axis_index("s")
        wid = sid * NC + cid
        rbase = sid * RPT

        pltpu.sync_copy(u_hbm.at[pl.ds(rbase, RPT)], stage)
        pltpu.sync_copy(stage, acc.at[pl.ds(rbase, RPT)])

        pltpu.sync_copy(src_hbm.at[pl.ds(wid * NCHUNK, NCHUNK)], sidx)
        pltpu.sync_copy(dst_hbm.at[pl.ds(wid * NCHUNK, NCHUNK)], didx)

        plsc.subcore_barrier()

        for b in range(NBUF):
            pltpu.async_copy(u_hbm.at[sidx.at[b]], rows[b], gs[b])

        def round_(g, c):
            for b in range(NBUF):
                j = g * NBUF + b
                pltpu.make_async_copy(u_hbm.at[sidx.at[j]], rows[b], gs[b]).wait()
                pltpu.async_copy(rows[b], acc.at[didx.at[j]], ss[b], add=True)
            for b in range(NBUF):
                pltpu.make_async_copy(rows[b], acc.at[didx.at[b]], ss[b]).wait()

                @pl.when(g + 1 < GRP)
                def _():
                    jn = (g + 1) * NBUF + b
                    pltpu.async_copy(u_hbm.at[sidx.at[jn]], rows[b], gs[b])

            return c

        lax.fori_loop(0, GRP, round_, 0)

        plsc.subcore_barrier()

        pltpu.sync_copy(acc.at[pl.ds(rbase, RPT)], stage)
        pltpu.sync_copy(stage, p_hbm.at[cid].at[pl.ds(rbase, RPT)])

    return agg


_agg8 = _make_agg(8)
_agg16 = _make_agg(16)
_agg40 = _make_agg(40)


_PAD = N_EXT - N


def _dinv(degp):
    deg = lax.dot_general(degp, jnp.ones((NW, 1), jnp.float32),
                          (((0,), (0,)), ((), ())),
                          preferred_element_type=jnp.float32)
    return lax.rsqrt(deg + 1.0)


def _tc_first_body(degp_ref, x_ref, w_ref, u_ref):
    dinv = _dinv(degp_ref[...])
    u_ref[pl.ds(0, N), :] = dinv[:N] * jnp.dot(
        x_ref[...], w_ref[...], preferred_element_type=jnp.float32)
    u_ref[pl.ds(N, _PAD), :] = jnp.zeros((_PAD, w_ref.shape[1]), jnp.float32)


def _tc_mid_body(degp_ref, p_ref, u_ref, b_ref, w_ref, un_ref):
    dinv = _dinv(degp_ref[...])
    s = p_ref[0] + p_ref[1] - u_ref[...]
    pre = dinv * s + b_ref[...]
    h = jnp.maximum(pre, 0.0)
    un_ref[...] = dinv * jnp.dot(h, w_ref[...],
                                 preferred_element_type=jnp.float32)
    un_ref[pl.ds(N, _PAD), :] = jnp.zeros((_PAD, w_ref.shape[1]), jnp.float32)


def _tc_last_body(degp_ref, p_ref, u_ref, b_ref, o_ref):
    dinv = _dinv(degp_ref[...])[:N]
    s = p_ref[0, pl.ds(0, N), :] + p_ref[1, pl.ds(0, N), :] - u_ref[pl.ds(0, N), :]
    pre = dinv * s + b_ref[...]
    m = jnp.max(pre, axis=1, keepdims=True)
    lse = jnp.log(jnp.sum(jnp.exp(pre - m), axis=1, keepdims=True)) + m
    o_ref[...] = pre - lse


def _tc_first(degp, x, w):
    return pl.pallas_call(
        _tc_first_body,
        out_shape=jax.ShapeDtypeStruct((N_EXT, w.shape[1]), jnp.float32),
    )(degp, x, w)


def _tc_mid(degp, p, u, b, w):
    return pl.pallas_call(
        _tc_mid_body,
        out_shape=jax.ShapeDtypeStruct((N_EXT, w.shape[1]), jnp.float32),
    )(degp, p, u, b, w)


def _tc_last(degp, p, u, b):
    return pl.pallas_call(
        _tc_last_body,
        out_shape=jax.ShapeDtypeStruct((N, u.shape[1]), jnp.float32),
    )(degp, p, u, b)


def kernel(x, adj, W1, b1, W2, b2, W3, b3, W4, b4):
    src = adj[0].astype(jnp.int32)
    dst = adj[1].astype(jnp.int32)
    pad = (jnp.arange(E_PAD - E, dtype=jnp.int32) % _PAD) + N
    src2 = jnp.concatenate([src, pad]).reshape(ROWS2D, CH)
    dst2 = jnp.concatenate([dst, pad]).reshape(ROWS2D, CH)

    degp = _deg(dst2)

    u1 = _tc_first(degp, x, W1)

    p1 = _agg8(u1, src2, dst2)
    u2 = _tc_mid(degp, p1, u1, b1.reshape(1, -1), W2)

    p2 = _agg16(u2, src2, dst2)
    u3 = _tc_mid(degp, p2, u2, b2.reshape(1, -1), W3)

    p3 = _agg8(u3, src2, dst2)
    u4 = _tc_mid(degp, p3, u3, b3.reshape(1, -1), W4)

    p4 = _agg40(u4, src2, dst2)
    return _tc_last(degp, p4, u4, b4.reshape(1, -1))


# speedup vs baseline: 51.5029x; 1.0491x over previous
import functools

import jax
import jax.numpy as jnp
from jax import lax
from jax.experimental import pallas as pl
from jax.experimental.pallas import tpu as pltpu
from jax.experimental.pallas import tpu_sc as plsc

N = 10000
N_EXT = 10112
E = 320000
NC, NS = 2, 16
NW = NC * NS
CH = 128
NCHUNK = 80
EPT = NCHUNK * CH
E_PAD = NW * EPT
ROWS2D = E_PAD // CH
RPT = N_EXT // NS
NBUF = 10
GRP = NCHUNK // NBUF

_MESH = plsc.VectorSubcoreMesh(core_axis_name="c", subcore_axis_name="s")


@functools.partial(
    pl.kernel,
    out_type=jax.ShapeDtypeStruct((NW, N_EXT), jnp.float32),
    mesh=_MESH,
    scratch_types=[
        pltpu.VMEM((N_EXT,), jnp.float32),
        pltpu.VMEM((NCHUNK, CH), jnp.int32),
    ],
    compiler_params=pltpu.CompilerParams(
        needs_layout_passes=False, use_tc_tiling_on_sc=False),
)
def _deg(dst_hbm, degp_hbm, deg_v, didx_v):
    cid = lax.axis_index("c")
    sid = lax.axis_index("s")
    wid = sid * NC + cid

    zeros = jnp.zeros((16,), jnp.float32)

    def zbody(i, c):
        deg_v[pl.ds(i * 16, 16)] = zeros
        return c

    lax.fori_loop(0, N_EXT // 16, zbody, 0)

    pltpu.sync_copy(dst_hbm.at[pl.ds(wid * NCHUNK, NCHUNK)], didx_v)

    ones = jnp.full((16,), 1.0, jnp.float32)

    def ebody(i, c):
        idx = didx_v[i // (CH // 16), pl.ds((i % (CH // 16)) * 16, 16)]
        plsc.addupdate_scatter(deg_v, [idx], ones)
        return c

    lax.fori_loop(0, EPT // 16, ebody, 0)

    pltpu.sync_copy(deg_v, degp_hbm.at[wid])


def _make_agg(F):
    @functools.partial(
        pl.kernel,
        out_type=jax.ShapeDtypeStruct((NC, N_EXT, F), jnp.float32),
        mesh=_MESH,
        scratch_types=(
            [
                pltpu.VMEM_SHARED((N_EXT, F), jnp.float32),
                pltpu.VMEM((NCHUNK, CH), jnp.int32),
                pltpu.VMEM((NCHUNK, CH), jnp.int32),
            ]
            + [pltpu.VMEM((CH, F), jnp.float32) for _ in range(NBUF)]
            + [pltpu.SemaphoreType.DMA for _ in range(2 * NBUF)]
        ),
        compiler_params=pltpu.CompilerParams(use_tc_tiling_on_sc=False),
    )
    def agg(u_hbm, src_hbm, dst_hbm, p_hbm, acc, sidx, didx, *bufs_sems):
        rows = bufs_sems[:NBUF]
        gs = bufs_sems[NBUF:2 * NBUF]
        ss = bufs_sems[2 * NBUF:]
        cid = lax.axis_index("c")
        sid = lax.axis_index("s")
        wid = sid * NC + cid
        rbase = sid * RPT

        pltpu.sync_copy(u_hbm.at[pl.ds(rbase, RPT)], acc.at[pl.ds(rbase, RPT)])

        pltpu.sync_copy(src_hbm.at[pl.ds(wid * NCHUNK, NCHUNK)], sidx)
        pltpu.sync_copy(dst_hbm.at[pl.ds(wid * NCHUNK, NCHUNK)], didx)

        plsc.subcore_barrier()

        for b in range(NBUF):
            pltpu.async_copy(u_hbm.at[sidx.at[b]], rows[b], gs[b])

        def round_(g, c):
            for b in range(NBUF):
                j = g * NBUF + b
                pltpu.make_async_copy(u_hbm.at[sidx.at[j]], rows[b], gs[b]).wait()
                pltpu.async_copy(rows[b], acc.at[didx.at[j]], ss[b], add=True)
            for b in range(NBUF):
                pltpu.make_async_copy(rows[b], acc.at[didx.at[b]], ss[b]).wait()

                @pl.when(g + 1 < GRP)
                def _():
                    jn = (g + 1) * NBUF + b
                    pltpu.async_copy(u_hbm.at[sidx.at[jn]], rows[b], gs[b])

            return c

        lax.fori_loop(0, GRP, round_, 0)

        plsc.subcore_barrier()

        pltpu.sync_copy(acc.at[pl.ds(rbase, RPT)], p_hbm.at[cid].at[pl.ds(rbase, RPT)])

    return agg


_agg8 = _make_agg(8)
_agg16 = _make_agg(16)
_agg40 = _make_agg(40)


_PAD = N_EXT - N


def _dinv(degp):
    deg = lax.dot_general(degp, jnp.ones((NW, 1), jnp.float32),
                          (((0,), (0,)), ((), ())),
                          preferred_element_type=jnp.float32)
    return lax.rsqrt(deg + 1.0)


def _tc_first_body(degp_ref, x_ref, w_ref, u_ref):
    dinv = _dinv(degp_ref[...])
    u_ref[pl.ds(0, N), :] = dinv[:N] * jnp.dot(
        x_ref[...], w_ref[...], preferred_element_type=jnp.float32)
    u_ref[pl.ds(N, _PAD), :] = jnp.zeros((_PAD, w_ref.shape[1]), jnp.float32)


def _tc_mid_body(degp_ref, p_ref, u_ref, b_ref, w_ref, un_ref):
    dinv = _dinv(degp_ref[...])
    s = p_ref[0] + p_ref[1] - u_ref[...]
    pre = dinv * s + b_ref[...]
    h = jnp.maximum(pre, 0.0)
    un_ref[...] = dinv * jnp.dot(h, w_ref[...],
                                 preferred_element_type=jnp.float32)
    un_ref[pl.ds(N, _PAD), :] = jnp.zeros((_PAD, w_ref.shape[1]), jnp.float32)


def _tc_last_body(degp_ref, p_ref, u_ref, b_ref, o_ref):
    dinv = _dinv(degp_ref[...])[:N]
    s = p_ref[0, pl.ds(0, N), :] + p_ref[1, pl.ds(0, N), :] - u_ref[pl.ds(0, N), :]
    pre = dinv * s + b_ref[...]
    m = jnp.max(pre, axis=1, keepdims=True)
    lse = jnp.log(jnp.sum(jnp.exp(pre - m), axis=1, keepdims=True)) + m
    o_ref[...] = pre - lse


def _tc_first(degp, x, w):
    return pl.pallas_call(
        _tc_first_body,
        out_shape=jax.ShapeDtypeStruct((N_EXT, w.shape[1]), jnp.float32),
    )(degp, x, w)


def _tc_mid(degp, p, u, b, w):
    return pl.pallas_call(
        _tc_mid_body,
        out_shape=jax.ShapeDtypeStruct((N_EXT, w.shape[1]), jnp.float32),
    )(degp, p, u, b, w)


def _tc_last(degp, p, u, b):
    return pl.pallas_call(
        _tc_last_body,
        out_shape=jax.ShapeDtypeStruct((N, u.shape[1]), jnp.float32),
    )(degp, p, u, b)


def kernel(x, adj, W1, b1, W2, b2, W3, b3, W4, b4):
    src = adj[0].astype(jnp.int32)
    dst = adj[1].astype(jnp.int32)
    pad = (jnp.arange(E_PAD - E, dtype=jnp.int32) % _PAD) + N
    src2 = jnp.concatenate([src, pad]).reshape(ROWS2D, CH)
    dst2 = jnp.concatenate([dst, pad]).reshape(ROWS2D, CH)

    degp = _deg(dst2)

    u1 = _tc_first(degp, x, W1)

    p1 = _agg8(u1, src2, dst2)
    u2 = _tc_mid(degp, p1, u1, b1.reshape(1, -1), W2)

    p2 = _agg16(u2, src2, dst2)
    u3 = _tc_mid(degp, p2, u2, b2.reshape(1, -1), W3)

    p3 = _agg8(u3, src2, dst2)
    u4 = _tc_mid(degp, p3, u3, b3.reshape(1, -1), W4)

    p4 = _agg40(u4, src2, dst2)
    return _tc_last(degp, p4, u4, b4.reshape(1, -1))


# speedup vs baseline: 53.1063x; 1.0311x over previous
import functools

import jax
import jax.numpy as jnp
from jax import lax
from jax.experimental import pallas as pl
from jax.experimental.pallas import tpu as pltpu
from jax.experimental.pallas import tpu_sc as plsc

N = 10000
N_EXT = 10112
E = 320000
NC, NS = 2, 16
NW = NC * NS
CH = 128
NCHUNK = 80
EPT = NCHUNK * CH
E_PAD = NW * EPT
ROWS2D = E_PAD // CH
RPT = N_EXT // NS
NBUF = 10
GRP = NCHUNK // NBUF

_MESH = plsc.VectorSubcoreMesh(core_axis_name="c", subcore_axis_name="s")


@functools.partial(
    pl.kernel,
    out_type=jax.ShapeDtypeStruct((NW, N_EXT), jnp.float32),
    mesh=_MESH,
    scratch_types=[
        pltpu.VMEM((N_EXT,), jnp.float32),
        pltpu.VMEM((NCHUNK, CH), jnp.int32),
    ],
    compiler_params=pltpu.CompilerParams(
        needs_layout_passes=False, use_tc_tiling_on_sc=False),
)
def _deg(dst_hbm, degp_hbm, deg_v, didx_v):
    cid = lax.axis_index("c")
    sid = lax.axis_index("s")
    wid = sid * NC + cid

    zeros = jnp.zeros((16,), jnp.float32)

    def zbody(i, c):
        deg_v[pl.ds(i * 16, 16)] = zeros
        return c

    lax.fori_loop(0, N_EXT // 16, zbody, 0)

    pltpu.sync_copy(dst_hbm.at[pl.ds(wid * NCHUNK, NCHUNK)], didx_v)

    ones = jnp.full((16,), 1.0, jnp.float32)

    def ebody(i, c):
        idx = didx_v[i // (CH // 16), pl.ds((i % (CH // 16)) * 16, 16)]
        plsc.addupdate_scatter(deg_v, [idx], ones)
        return c

    lax.fori_loop(0, EPT // 16, ebody, 0)

    pltpu.sync_copy(deg_v, degp_hbm.at[wid])


def _make_agg(F):
    @functools.partial(
        pl.kernel,
        out_type=jax.ShapeDtypeStruct((NC, N_EXT, F), jnp.float32),
        mesh=_MESH,
        scratch_types=(
            [
                pltpu.VMEM_SHARED((N_EXT, F), jnp.float32),
                pltpu.VMEM((NCHUNK, CH), jnp.int32),
                pltpu.VMEM((NCHUNK, CH), jnp.int32),
            ]
            + [pltpu.VMEM((CH, F), jnp.float32) for _ in range(NBUF)]
            + [pltpu.SemaphoreType.DMA for _ in range(2 * NBUF)]
        ),
        compiler_params=pltpu.CompilerParams(use_tc_tiling_on_sc=False),
    )
    def agg(u_hbm, src_hbm, dst_hbm, p_hbm, acc, sidx, didx, *bufs_sems):
        rows = bufs_sems[:NBUF]
        gs = bufs_sems[NBUF:2 * NBUF]
        ss = bufs_sems[2 * NBUF:]
        cid = lax.axis_index("c")
        sid = lax.axis_index("s")
        wid = sid * NC + cid
        rbase = sid * RPT

        pltpu.sync_copy(u_hbm.at[pl.ds(rbase, RPT)], acc.at[pl.ds(rbase, RPT)])

        pltpu.sync_copy(src_hbm.at[pl.ds(wid * NCHUNK, NCHUNK)], sidx)
        pltpu.sync_copy(dst_hbm.at[pl.ds(wid * NCHUNK, NCHUNK)], didx)

        plsc.subcore_barrier()

        for b in range(NBUF):
            pltpu.async_copy(u_hbm.at[sidx.at[b]], rows[b], gs[b])

        def round_(g, c):
            for b in range(NBUF):
                j = g * NBUF + b
                pltpu.make_async_copy(u_hbm.at[sidx.at[j]], rows[b], gs[b]).wait()
                pltpu.async_copy(rows[b], acc.at[didx.at[j]], ss[b], add=True)
            for b in range(NBUF):
                pltpu.make_async_copy(rows[b], acc.at[didx.at[b]], ss[b]).wait()

                @pl.when(g + 1 < GRP)
                def _():
                    jn = (g + 1) * NBUF + b
                    pltpu.async_copy(u_hbm.at[sidx.at[jn]], rows[b], gs[b])

            return c

        lax.fori_loop(0, GRP, round_, 0)

        plsc.subcore_barrier()

        pltpu.sync_copy(acc.at[pl.ds(rbase, RPT)], p_hbm.at[cid].at[pl.ds(rbase, RPT)])

    return agg


_agg8 = _make_agg(8)
_agg16 = _make_agg(16)
_agg40 = _make_agg(40)


_PAD = N_EXT - N


def _tc_first_body(degp_ref, x_ref, w_ref, dinv_ref, u_ref):
    deg = jnp.dot(jnp.ones((1, NW), jnp.float32), degp_ref[...],
                  preferred_element_type=jnp.float32)
    col = lax.broadcasted_iota(jnp.int32, (1, N_EXT), 1)
    dinv = jnp.where(col < N, lax.rsqrt(deg + 1.0), 0.0)
    dinv_ref[...] = dinv
    h = lax.dot_general(w_ref[...], x_ref[...], (((1,), (1,)), ((), ())),
                        preferred_element_type=jnp.float32)
    full = jnp.concatenate(
        [h, jnp.zeros((w_ref.shape[0], _PAD), jnp.float32)], axis=1)
    u_ref[...] = dinv * full


def _tc_mid_body(dinv_ref, p_ref, u_ref, b_ref, w_ref, un_ref):
    dinv = dinv_ref[...]
    s = p_ref[0] + p_ref[1] - u_ref[...]
    pre = dinv * s + b_ref[...]
    h = jnp.maximum(pre, 0.0)
    un_ref[...] = dinv * jnp.dot(w_ref[...], h,
                                 preferred_element_type=jnp.float32)


def _tc_last_body(dinv_ref, p_ref, u_ref, b_ref, o_ref):
    dinv = dinv_ref[...][:, :N]
    s = p_ref[0, :, pl.ds(0, N)] + p_ref[1, :, pl.ds(0, N)] - u_ref[:, pl.ds(0, N)]
    pre = dinv * s + b_ref[...]
    m = jnp.max(pre, axis=0, keepdims=True)
    lse = jnp.log(jnp.sum(jnp.exp(pre - m), axis=0, keepdims=True)) + m
    o_ref[...] = pre - lse


def _tc_first(degp, x, wT):
    return pl.pallas_call(
        _tc_first_body,
        out_shape=(
            jax.ShapeDtypeStruct((1, N_EXT), jnp.float32),
            jax.ShapeDtypeStruct((wT.shape[0], N_EXT), jnp.float32),
        ),
    )(degp, x, wT)


def _tc_mid(dinv, pT, uT, bT, wT):
    return pl.pallas_call(
        _tc_mid_body,
        out_shape=jax.ShapeDtypeStruct((wT.shape[0], N_EXT), jnp.float32),
    )(dinv, pT, uT, bT, wT)


def _tc_last(dinv, pT, uT, bT):
    return pl.pallas_call(
        _tc_last_body,
        out_shape=jax.ShapeDtypeStruct((uT.shape[0], N), jnp.float32),
    )(dinv, pT, uT, bT)


def kernel(x, adj, W1, b1, W2, b2, W3, b3, W4, b4):
    src = adj[0].astype(jnp.int32)
    dst = adj[1].astype(jnp.int32)
    pad = (jnp.arange(E_PAD - E, dtype=jnp.int32) % _PAD) + N
    src2 = jnp.concatenate([src, pad]).reshape(ROWS2D, CH)
    dst2 = jnp.concatenate([dst, pad]).reshape(ROWS2D, CH)

    degp = _deg(dst2)

    dinv, u1T = _tc_first(degp, x, W1.T)

    p1 = _agg8(u1T.T, src2, dst2)
    u2T = _tc_mid(dinv, p1.transpose(0, 2, 1), u1T, b1.reshape(-1, 1), W2.T)

    p2 = _agg16(u2T.T, src2, dst2)
    u3T = _tc_mid(dinv, p2.transpose(0, 2, 1), u2T, b2.reshape(-1, 1), W3.T)

    p3 = _agg8(u3T.T, src2, dst2)
    u4T = _tc_mid(dinv, p3.transpose(0, 2, 1), u3T, b3.reshape(-1, 1), W4.T)

    p4 = _agg40(u4T.T, src2, dst2)
    outT = _tc_last(dinv, p4.transpose(0, 2, 1), u4T, b4.reshape(-1, 1))
    return outT.T


# speedup vs baseline: 58.6744x; 1.1048x over previous
import functools

import jax
import jax.numpy as jnp
from jax import lax
from jax.experimental import pallas as pl
from jax.experimental.pallas import tpu as pltpu
from jax.experimental.pallas import tpu_sc as plsc

N = 10000
N_EXT = 10112
E = 320000
NC, NS = 2, 16
NW = NC * NS
CH = 128
NCHUNK = 80
EPT = NCHUNK * CH
E_PAD = NW * EPT
ROWS2D = E_PAD // CH
RPT = N_EXT // NS
NBUF = 10
GRP = NCHUNK // NBUF

_MESH = plsc.VectorSubcoreMesh(core_axis_name="c", subcore_axis_name="s")


@functools.partial(
    pl.kernel,
    out_type=jax.ShapeDtypeStruct((NW, N_EXT), jnp.float32),
    mesh=_MESH,
    scratch_types=[
        pltpu.VMEM((N_EXT,), jnp.float32),
        pltpu.VMEM((NCHUNK, CH), jnp.int32),
    ],
    compiler_params=pltpu.CompilerParams(
        needs_layout_passes=False, use_tc_tiling_on_sc=False),
)
def _deg(dst_hbm, degp_hbm, deg_v, didx_v):
    cid = lax.axis_index("c")
    sid = lax.axis_index("s")
    wid = sid * NC + cid

    zeros = jnp.zeros((16,), jnp.float32)

    def zbody(i, c):
        deg_v[pl.ds(i * 16, 16)] = zeros
        return c

    lax.fori_loop(0, N_EXT // 16, zbody, 0)

    pltpu.sync_copy(dst_hbm.at[pl.ds(wid * NCHUNK, NCHUNK)], didx_v)

    ones = jnp.full((16,), 1.0, jnp.float32)

    def ebody(i, c):
        idx = didx_v[i // (CH // 16), pl.ds((i % (CH // 16)) * 16, 16)]
        plsc.addupdate_scatter(deg_v, [idx], ones)
        return c

    lax.fori_loop(0, EPT // 16, ebody, 0)

    pltpu.sync_copy(deg_v, degp_hbm.at[wid])


def _make_agg(F):
    @functools.partial(
        pl.kernel,
        out_type=jax.ShapeDtypeStruct((N_EXT, 2 * F), jnp.float32),
        mesh=_MESH,
        scratch_types=(
            [
                pltpu.VMEM_SHARED((N_EXT, F), jnp.float32),
                pltpu.VMEM((NCHUNK, CH), jnp.int32),
                pltpu.VMEM((NCHUNK, CH), jnp.int32),
            ]
            + [pltpu.VMEM((CH, F), jnp.float32) for _ in range(NBUF)]
            + [pltpu.SemaphoreType.DMA for _ in range(2 * NBUF)]
        ),
        compiler_params=pltpu.CompilerParams(use_tc_tiling_on_sc=False),
    )
    def agg(u_hbm, src_hbm, dst_hbm, p_hbm, acc, sidx, didx, *bufs_sems):
        rows = bufs_sems[:NBUF]
        gs = bufs_sems[NBUF:2 * NBUF]
        ss = bufs_sems[2 * NBUF:]
        cid = lax.axis_index("c")
        sid = lax.axis_index("s")
        wid = sid * NC + cid
        rbase = sid * RPT

        pltpu.sync_copy(u_hbm.at[pl.ds(rbase, RPT)], acc.at[pl.ds(rbase, RPT)])

        pltpu.sync_copy(src_hbm.at[pl.ds(wid * NCHUNK, NCHUNK)], sidx)
        pltpu.sync_copy(dst_hbm.at[pl.ds(wid * NCHUNK, NCHUNK)], didx)

        plsc.subcore_barrier()

        for b in range(NBUF):
            pltpu.async_copy(u_hbm.at[sidx.at[b]], rows[b], gs[b])

        def round_(g, c):
            for b in range(NBUF):
                j = g * NBUF + b
                pltpu.make_async_copy(u_hbm.at[sidx.at[j]], rows[b], gs[b]).wait()
                pltpu.async_copy(rows[b], acc.at[didx.at[j]], ss[b], add=True)
            for b in range(NBUF):
                pltpu.make_async_copy(rows[b], acc.at[didx.at[b]], ss[b]).wait()

                @pl.when(g + 1 < GRP)
                def _():
                    jn = (g + 1) * NBUF + b
                    pltpu.async_copy(u_hbm.at[sidx.at[jn]], rows[b], gs[b])

            return c

        lax.fori_loop(0, GRP, round_, 0)

        plsc.subcore_barrier()

        pltpu.sync_copy(acc.at[pl.ds(rbase, RPT)],
                        p_hbm.at[pl.ds(rbase, RPT), pl.ds(cid * F, F)])

    return agg


_agg8 = _make_agg(8)
_agg16 = _make_agg(16)
_agg40 = _make_agg(40)


_PAD = N_EXT - N


def _tc_first_body(degp_ref, x_ref, w_ref, dinv_ref, u_ref):
    deg = jnp.dot(jnp.ones((1, NW), jnp.float32), degp_ref[...],
                  preferred_element_type=jnp.float32)
    col = lax.broadcasted_iota(jnp.int32, (1, N_EXT), 1)
    dinv = jnp.where(col < N, lax.rsqrt(deg + 1.0), 0.0)
    dinv_ref[...] = dinv
    h = lax.dot_general(w_ref[...], x_ref[...], (((1,), (1,)), ((), ())),
                        preferred_element_type=jnp.float32)
    full = jnp.concatenate(
        [h, jnp.zeros((w_ref.shape[0], _PAD), jnp.float32)], axis=1)
    u_ref[...] = dinv * full


def _tc_mid_body(dinv_ref, p_ref, u_ref, b_ref, w_ref, un_ref):
    dinv = dinv_ref[...]
    fv = u_ref.shape[0]
    p = p_ref[...]
    s = p[:fv] + p[fv:] - u_ref[...]
    pre = dinv * s + b_ref[...]
    h = jnp.maximum(pre, 0.0)
    un_ref[...] = dinv * jnp.dot(w_ref[...], h,
                                 preferred_element_type=jnp.float32)


def _tc_last_body(dinv_ref, p_ref, u_ref, b_ref, o_ref):
    dinv = dinv_ref[...][:, :N]
    fv = u_ref.shape[0]
    p = p_ref[:, pl.ds(0, N)]
    s = p[:fv] + p[fv:] - u_ref[:, pl.ds(0, N)]
    pre = dinv * s + b_ref[...]
    m = jnp.max(pre, axis=0, keepdims=True)
    lse = jnp.log(jnp.sum(jnp.exp(pre - m), axis=0, keepdims=True)) + m
    o_ref[...] = pre - lse


def _tc_first(degp, x, wT):
    return pl.pallas_call(
        _tc_first_body,
        out_shape=(
            jax.ShapeDtypeStruct((1, N_EXT), jnp.float32),
            jax.ShapeDtypeStruct((wT.shape[0], N_EXT), jnp.float32),
        ),
    )(degp, x, wT)


def _tc_mid(dinv, pT, uT, bT, wT):
    return pl.pallas_call(
        _tc_mid_body,
        out_shape=jax.ShapeDtypeStruct((wT.shape[0], N_EXT), jnp.float32),
    )(dinv, pT, uT, bT, wT)


def _tc_last(dinv, pT, uT, bT):
    return pl.pallas_call(
        _tc_last_body,
        out_shape=jax.ShapeDtypeStruct((uT.shape[0], N), jnp.float32),
    )(dinv, pT, uT, bT)


def kernel(x, adj, W1, b1, W2, b2, W3, b3, W4, b4):
    src = adj[0].astype(jnp.int32)
    dst = adj[1].astype(jnp.int32)
    pad = (jnp.arange(E_PAD - E, dtype=jnp.int32) % _PAD) + N
    src2 = jnp.concatenate([src, pad]).reshape(ROWS2D, CH)
    dst2 = jnp.concatenate([dst, pad]).reshape(ROWS2D, CH)

    degp = _deg(dst2)

    dinv, u1T = _tc_first(degp, x, W1.T)

    p1 = _agg8(u1T.T, src2, dst2)
    u2T = _tc_mid(dinv, p1.T, u1T, b1.reshape(-1, 1), W2.T)

    p2 = _agg16(u2T.T, src2, dst2)
    u3T = _tc_mid(dinv, p2.T, u2T, b2.reshape(-1, 1), W3.T)

    p3 = _agg8(u3T.T, src2, dst2)
    u4T = _tc_mid(dinv, p3.T, u3T, b3.reshape(-1, 1), W4.T)

    p4 = _agg40(u4T.T, src2, dst2)
    outT = _tc_last(dinv, p4.T, u4T, b4.reshape(-1, 1))
    return outT.T
